# baseline (speedup 1.0000x reference)
import functools

import jax
import jax.numpy as jnp
from jax import lax
from jax.experimental import pallas as pl
from jax.experimental.pallas import tpu as pltpu
from jax.experimental.pallas import tpu_sc as plsc

NUM_EMB = 1000000
D = 64
BATCH = 4096
SEQ = 200
B_TOTAL = BATCH * SEQ
NC = 2
NS = 16
NW = NC * NS
BPW = B_TOTAL // NW
CHUNK = 512
NCHUNK = BPW // CHUNK
STREAM = 512
K = CHUNK // STREAM
NBUF = 2
NGROUP = NCHUNK // NBUF


def _embed_body(idx_hbm, table_hbm, out_hbm, idx_v, rows_v, gsems, ssems):
    wid = lax.axis_index("s") * NC + lax.axis_index("c")
    base = wid * BPW
    pltpu.sync_copy(idx_hbm.at[pl.ds(base, BPW)], idx_v)

    def fire_gather(chunk, b):
        off = chunk * CHUNK
        for j in range(K):
            pltpu.async_copy(
                table_hbm.at[idx_v.at[pl.ds(off + j * STREAM, STREAM)]],
                rows_v.at[b, pl.ds(j * STREAM, STREAM)],
                gsems[b],
            )

    def wait_gather(b):
        pltpu.make_async_copy(
            table_hbm.at[pl.ds(0, CHUNK)], rows_v.at[b], gsems[b]
        ).wait()

    def fire_store(chunk, b):
        pltpu.async_copy(
            rows_v.at[b], out_hbm.at[pl.ds(base + chunk * CHUNK, CHUNK)], ssems[b]
        )

    def wait_store(b):
        pltpu.make_async_copy(
            rows_v.at[b], out_hbm.at[pl.ds(0, CHUNK)], ssems[b]
        ).wait()

    for b in range(NBUF):
        fire_gather(b, b)

    def group(g, carry):
        for b in range(NBUF):
            i = g * NBUF + b
            wait_gather(b)
            fire_store(i, b)
            wait_store(b)
            fire_gather(i + NBUF, b)
        return carry

    lax.fori_loop(0, NGROUP - 1, group, 0)

    for b in range(NBUF):
        i = (NGROUP - 1) * NBUF + b
        wait_gather(b)
        fire_store(i, b)
    for b in range(NBUF):
        wait_store(b)


@jax.jit
def _embed(inputs_flat, embedding):
    mesh = plsc.VectorSubcoreMesh(
        core_axis_name="c", subcore_axis_name="s", num_cores=NC, num_subcores=NS
    )
    return pl.kernel(
        _embed_body,
        out_type=jax.ShapeDtypeStruct((B_TOTAL, D), jnp.float32),
        mesh=mesh,
        scratch_types=[
            pltpu.VMEM((BPW,), jnp.int32),
            pltpu.VMEM((NBUF, CHUNK, D), jnp.float32),
            [pltpu.SemaphoreType.DMA] * NBUF,
            [pltpu.SemaphoreType.DMA] * NBUF,
        ],
        compiler_params=pltpu.CompilerParams(use_tc_tiling_on_sc=False),
    )(inputs_flat, embedding)


def kernel(inputs, embedding):
    out = _embed(inputs.reshape(-1), embedding)
    return out.reshape(BATCH, SEQ, D)


# speedup vs baseline: 1.3298x; 1.3298x over previous
import functools

import jax
import jax.numpy as jnp
from jax import lax
from jax.experimental import pallas as pl
from jax.experimental.pallas import tpu as pltpu
from jax.experimental.pallas import tpu_sc as plsc

NUM_EMB = 1000000
D = 64
BATCH = 4096
SEQ = 200
B_TOTAL = BATCH * SEQ
NC = 2
NS = 16
NW = NC * NS
BPW = B_TOTAL // NW
CHUNK = 512
NCHUNK = BPW // CHUNK
STREAM = 512
K = CHUNK // STREAM
NBUF = 2
NGROUP = NCHUNK // NBUF


def _embed_body(idx_hbm, table_hbm, out_hbm, idx_v, rows_v, gsems, ssems):
    wid = lax.axis_index("s") * NC + lax.axis_index("c")
    base = wid * BPW
    pltpu.sync_copy(idx_hbm.at[pl.ds(base, BPW)], idx_v)

    def fire_gather(chunk, b):
        off = chunk * CHUNK
        for j in range(K):
            pltpu.async_copy(
                table_hbm.at[idx_v.at[pl.ds(off + j * STREAM, STREAM)]],
                rows_v.at[b, pl.ds(j * STREAM, STREAM)],
                gsems[b],
            )

    def wait_gather(b):
        pltpu.make_async_copy(
            table_hbm.at[pl.ds(0, CHUNK)], rows_v.at[b], gsems[b]
        ).wait()

    def fire_store(chunk, b):
        pltpu.async_copy(
            rows_v.at[b],
            out_hbm.at[pl.ds(base + chunk * CHUNK, CHUNK), pl.ds(0, D)],
            ssems[b],
        )

    def wait_store(b):
        pltpu.make_async_copy(
            rows_v.at[b], out_hbm.at[pl.ds(0, CHUNK), pl.ds(0, D)], ssems[b]
        ).wait()

    for b in range(NBUF):
        fire_gather(b, b)

    def group(g, carry):
        for b in range(NBUF):
            i = g * NBUF + b
            wait_gather(b)
            fire_store(i, b)
            wait_store(b)
            fire_gather(i + NBUF, b)
        return carry

    lax.fori_loop(0, NGROUP - 1, group, 0)

    for b in range(NBUF):
        i = (NGROUP - 1) * NBUF + b
        wait_gather(b)
        fire_store(i, b)
    for b in range(NBUF):
        wait_store(b)


@jax.jit
def _embed(inputs_flat, embedding):
    mesh = plsc.VectorSubcoreMesh(
        core_axis_name="c", subcore_axis_name="s", num_cores=NC, num_subcores=NS
    )
    return pl.kernel(
        _embed_body,
        out_type=jax.ShapeDtypeStruct((B_TOTAL, 128), jnp.float32),
        mesh=mesh,
        scratch_types=[
            pltpu.VMEM((BPW,), jnp.int32),
            pltpu.VMEM((NBUF, CHUNK, D), jnp.float32),
            [pltpu.SemaphoreType.DMA] * NBUF,
            [pltpu.SemaphoreType.DMA] * NBUF,
        ],
        compiler_params=pltpu.CompilerParams(use_tc_tiling_on_sc=False),
    )(inputs_flat, embedding)


def kernel(inputs, embedding):
    out = _embed(inputs.reshape(-1), embedding)
    return out[:, :D].reshape(BATCH, SEQ, D)
